# Initial kernel scaffold; baseline (speedup 1.0000x reference)
import jax
import jax.numpy as jnp
from jax.experimental import pallas as pl


def kernel(outputs, masks):
    raise NotImplementedError("write your pallas kernel here")


# speedup vs baseline: 23.1871x; 23.1871x over previous
import functools

import jax
import jax.numpy as jnp
from jax import lax
from jax.experimental import pallas as pl
from jax.experimental.pallas import tpu as pltpu
from jax.experimental.pallas import tpu_sc as plsc

L = 16
NC, NS = 2, 16
NTEC = NC * NS
K = 2048
NEL = 16 * 512 * 512
PER = NEL // NTEC
CHUNK = 4096
NCH = PER // CHUNK
NV = CHUNK // L


def _sc_histogram_body(x_hbm, m_hbm, cnt_out, sum_out, g_out, p_out,
                       xbuf, mbuf, cnt_h, sum_h, red_c, red_s, gtmp, ptmp):
    wid = lax.axis_index("s") * NC + lax.axis_index("c")
    lane = lax.iota(jnp.int32, L)
    zeros16 = jnp.zeros((L,), jnp.float32)
    ones16 = jnp.ones((L,), jnp.float32)

    def zbody(i, _):
        cnt_h[pl.ds(i * L, L)] = zeros16
        sum_h[pl.ds(i * L, L)] = zeros16
        return _
    lax.fori_loop(0, L * K // L, zbody, None)
    lanebase = lane * K

    base0 = wid * PER

    def chunk_body(c, carry):
        pltpu.sync_copy(x_hbm.at[pl.ds(base0 + c * CHUNK, CHUNK)], xbuf)
        pltpu.sync_copy(m_hbm.at[pl.ds(base0 + c * CHUNK, CHUNK)], mbuf)

        def vbody(i, carry2):
            ag, ap = carry2
            x = xbuf[pl.ds(i * L, L)]
            mi = mbuf[pl.ds(i * L, L)]
            mf = mi.astype(jnp.float32)
            s = 1.0 / (1.0 + jnp.exp(-x))
            ag = ag + mf
            ap = ap + mf * (1.0 - s)
            b = jnp.minimum((s * float(K)).astype(jnp.int32), K - 1)
            idx = lanebase + b
            neg = mi == 0
            plsc.addupdate_scatter(cnt_h, [idx], ones16, mask=neg)
            plsc.addupdate_scatter(sum_h, [idx], 1.0 + s, mask=neg)
            return ag, ap

        return lax.fori_loop(0, NV, vbody, carry)

    accg, accp = lax.fori_loop(0, NCH, chunk_body, (zeros16, zeros16))

    def rbody(i, _):
        ac = cnt_h[pl.ds(i * L, L)]
        as_ = sum_h[pl.ds(i * L, L)]
        for l in range(1, L):
            ac = ac + cnt_h[pl.ds(l * K + i * L, L)]
            as_ = as_ + sum_h[pl.ds(l * K + i * L, L)]
        red_c[pl.ds(i * L, L)] = ac
        red_s[pl.ds(i * L, L)] = as_
        return _
    lax.fori_loop(0, K // L, rbody, None)

    gtmp[...] = accg
    ptmp[...] = accp
    pltpu.sync_copy(red_c, cnt_out.at[wid])
    pltpu.sync_copy(red_s, sum_out.at[wid])
    pltpu.sync_copy(gtmp, g_out.at[wid])
    pltpu.sync_copy(ptmp, p_out.at[wid])


@functools.cache
def _sc_histogram():
    return pl.kernel(
        _sc_histogram_body,
        out_type=(
            jax.ShapeDtypeStruct((NTEC, K), jnp.float32),
            jax.ShapeDtypeStruct((NTEC, K), jnp.float32),
            jax.ShapeDtypeStruct((NTEC, L), jnp.float32),
            jax.ShapeDtypeStruct((NTEC, L), jnp.float32),
        ),
        mesh=plsc.VectorSubcoreMesh(
            core_axis_name="c", subcore_axis_name="s",
            num_cores=NC, num_subcores=NS),
        compiler_params=pltpu.CompilerParams(needs_layout_passes=False),
        scratch_types=[
            pltpu.VMEM((CHUNK,), jnp.float32),
            pltpu.VMEM((CHUNK,), jnp.int32),
            pltpu.VMEM((L * K,), jnp.float32),
            pltpu.VMEM((L * K,), jnp.float32),
            pltpu.VMEM((K,), jnp.float32),
            pltpu.VMEM((K,), jnp.float32),
            pltpu.VMEM((L,), jnp.float32),
            pltpu.VMEM((L,), jnp.float32),
        ],
    )


def _tc_finish_body(cnt_ref, sum_ref, g_ref, p_ref, out_ref):
    cnt1 = jnp.sum(cnt_ref[...], axis=0, keepdims=True)
    sum1 = jnp.sum(sum_ref[...], axis=0, keepdims=True)
    R, C = K // 128, 128
    cnt2d = jnp.concatenate([cnt1[:, i * C:(i + 1) * C] for i in range(R)],
                            axis=0)
    sum2d = jnp.concatenate([sum1[:, i * C:(i + 1) * C] for i in range(R)],
                            axis=0)
    G = jnp.sum(g_ref[...])
    P = jnp.sum(p_ref[...])

    kk = lax.broadcasted_iota(jnp.int32, (C, C), 0)
    jj = lax.broadcasted_iota(jnp.int32, (C, C), 1)
    A = (kk >= jj).astype(jnp.float32)
    inrow = jnp.dot(cnt2d, A, preferred_element_type=jnp.float32)
    rowsum = jnp.sum(cnt2d, axis=1, keepdims=True)
    rr = lax.broadcasted_iota(jnp.int32, (R, R), 0)
    cc = lax.broadcasted_iota(jnp.int32, (R, R), 1)
    B = (cc > rr).astype(jnp.float32)
    ra = jnp.dot(B, rowsum, preferred_element_type=jnp.float32)
    r_hi = inrow + ra
    r_lo = r_hi - cnt2d

    denom = (G + r_lo) * (G + r_hi)
    W = jnp.where(cnt2d > 0, G * cnt2d / jnp.maximum(denom, 1.0), 0.0)
    W = W + jnp.where((G == 0.0) & (r_lo == 0.0) & (cnt2d > 0), 1.0, 0.0)
    ebar = sum2d / jnp.maximum(cnt2d, 1.0)
    loss = P * (1.0 / float(NEL)) + jnp.sum(ebar * W)
    out_ref[0, 0] = loss


def _tc_finish(cnt, sm, g, p):
    return pl.pallas_call(
        _tc_finish_body,
        out_shape=jax.ShapeDtypeStruct((1, 1), jnp.float32),
        in_specs=[pl.BlockSpec(memory_space=pltpu.VMEM)] * 4,
        out_specs=pl.BlockSpec(memory_space=pltpu.SMEM),
    )(cnt, sm, g, p)


def kernel(outputs, masks):
    x = outputs.reshape(-1)
    m = masks.reshape(-1).astype(jnp.int32)
    cnt, sm, g, p = _sc_histogram()(x, m)
    return _tc_finish(cnt, sm, g, p).reshape(())


# speedup vs baseline: 27.2736x; 1.1762x over previous
import functools

import jax
import jax.numpy as jnp
from jax import lax
from jax.experimental import pallas as pl
from jax.experimental.pallas import tpu as pltpu
from jax.experimental.pallas import tpu_sc as plsc

L = 16
NC, NS = 2, 16
NTEC = NC * NS
K = 2048
NEL = 16 * 512 * 512
PER = NEL // NTEC
CHUNK = 4096
NCH = PER // CHUNK
NV = CHUNK // L
UNROLL = 8


def _sc_histogram_body(x_hbm, m_hbm, cnt_out, sum_out, s_out,
                       xb0, mb0, xb1, mb1, cnt_h, sum_h,
                       red_c, red_s, stmp, sx0, sm0, sx1, sm1):
    wid = lax.axis_index("s") * NC + lax.axis_index("c")
    lane = lax.iota(jnp.int32, L)
    lanebase = lane * K
    zeros16 = jnp.zeros((L,), jnp.float32)
    ones16 = jnp.ones((L,), jnp.float32)

    def zbody(i, _):
        cnt_h[pl.ds(i * L, L)] = zeros16
        sum_h[pl.ds(i * L, L)] = zeros16
        return _
    lax.fori_loop(0, K, zbody, None)

    base0 = wid * PER

    def start_fetch(c, xb, mb, sx, sm_):
        pltpu.async_copy(x_hbm.at[pl.ds(base0 + c * CHUNK, CHUNK)], xb, sx)
        pltpu.async_copy(m_hbm.at[pl.ds(base0 + c * CHUNK, CHUNK)], mb, sm_)

    def wait_fetch(c, xb, mb, sx, sm_):
        pltpu.make_async_copy(
            x_hbm.at[pl.ds(base0 + c * CHUNK, CHUNK)], xb, sx).wait()
        pltpu.make_async_copy(
            m_hbm.at[pl.ds(base0 + c * CHUNK, CHUNK)], mb, sm_).wait()

    def process(xb, mb, acc):
        def vbody(j, a):
            for r in range(UNROLL):
                off = (j * UNROLL + r) * L
                x = xb[pl.ds(off, L)]
                mi = mb[pl.ds(off, L)]
                s = 1.0 / (1.0 + jnp.exp(-x))
                a = a + s
                b = jnp.minimum((s * float(K)).astype(jnp.int32), K - 1)
                neg = mi == 0
                plsc.addupdate_scatter(cnt_h, [lanebase + b], ones16, mask=neg)
                plsc.addupdate_scatter(sum_h, [lanebase + b], 1.0 + s,
                                       mask=neg)
            return a
        return lax.fori_loop(0, NV // UNROLL, vbody, acc)

    start_fetch(0, xb0, mb0, sx0, sm0)
    start_fetch(1, xb1, mb1, sx1, sm1)

    def chunk_body(i, acc):
        c0 = 2 * i
        wait_fetch(c0, xb0, mb0, sx0, sm0)
        acc = process(xb0, mb0, acc)

        @pl.when(c0 + 2 < NCH)
        def _():
            start_fetch(c0 + 2, xb0, mb0, sx0, sm0)

        wait_fetch(c0 + 1, xb1, mb1, sx1, sm1)
        acc = process(xb1, mb1, acc)

        @pl.when(c0 + 3 < NCH)
        def _():
            start_fetch(c0 + 3, xb1, mb1, sx1, sm1)

        return acc

    accs = lax.fori_loop(0, NCH // 2, chunk_body, zeros16)

    def rbody(i, _):
        ac = cnt_h[pl.ds(i * L, L)]
        as_ = sum_h[pl.ds(i * L, L)]
        for l in range(1, L):
            ac = ac + cnt_h[pl.ds(l * K + i * L, L)]
            as_ = as_ + sum_h[pl.ds(l * K + i * L, L)]
        red_c[pl.ds(i * L, L)] = ac
        red_s[pl.ds(i * L, L)] = as_
        return _
    lax.fori_loop(0, K // L, rbody, None)

    stmp[...] = accs
    pltpu.sync_copy(red_c, cnt_out.at[wid])
    pltpu.sync_copy(red_s, sum_out.at[wid])
    pltpu.sync_copy(stmp, s_out.at[wid])


@functools.cache
def _sc_histogram():
    return pl.kernel(
        _sc_histogram_body,
        out_type=(
            jax.ShapeDtypeStruct((NTEC, K), jnp.float32),
            jax.ShapeDtypeStruct((NTEC, K), jnp.float32),
            jax.ShapeDtypeStruct((NTEC, L), jnp.float32),
        ),
        mesh=plsc.VectorSubcoreMesh(
            core_axis_name="c", subcore_axis_name="s",
            num_cores=NC, num_subcores=NS),
        compiler_params=pltpu.CompilerParams(needs_layout_passes=False),
        scratch_types=[
            pltpu.VMEM((CHUNK,), jnp.float32),
            pltpu.VMEM((CHUNK,), jnp.int32),
            pltpu.VMEM((CHUNK,), jnp.float32),
            pltpu.VMEM((CHUNK,), jnp.int32),
            pltpu.VMEM((L * K,), jnp.float32),
            pltpu.VMEM((L * K,), jnp.float32),
            pltpu.VMEM((K,), jnp.float32),
            pltpu.VMEM((K,), jnp.float32),
            pltpu.VMEM((L,), jnp.float32),
            pltpu.SemaphoreType.DMA,
            pltpu.SemaphoreType.DMA,
            pltpu.SemaphoreType.DMA,
            pltpu.SemaphoreType.DMA,
        ],
    )


def _tc_finish_body(cnt_ref, sum_ref, s_ref, out_ref):
    cnt1 = jnp.sum(cnt_ref[...], axis=0, keepdims=True)
    sum1 = jnp.sum(sum_ref[...], axis=0, keepdims=True)
    R, C = K // 128, 128
    cnt2d = jnp.concatenate([cnt1[:, i * C:(i + 1) * C] for i in range(R)],
                            axis=0)
    sum2d = jnp.concatenate([sum1[:, i * C:(i + 1) * C] for i in range(R)],
                            axis=0)
    N = float(NEL)
    M = jnp.sum(cnt2d)
    Stot = jnp.sum(sum2d)
    Sall = jnp.sum(s_ref[...])
    G = N - M
    P = N - Sall - 2.0 * M + Stot

    kk = lax.broadcasted_iota(jnp.int32, (C, C), 0)
    jj = lax.broadcasted_iota(jnp.int32, (C, C), 1)
    A = (kk >= jj).astype(jnp.float32)
    inrow = jnp.dot(cnt2d, A, preferred_element_type=jnp.float32)
    rowsum = jnp.sum(cnt2d, axis=1, keepdims=True)
    rr = lax.broadcasted_iota(jnp.int32, (R, R), 0)
    cc = lax.broadcasted_iota(jnp.int32, (R, R), 1)
    B = (cc > rr).astype(jnp.float32)
    ra = jnp.dot(B, rowsum, preferred_element_type=jnp.float32)
    r_hi = inrow + ra
    r_lo = r_hi - cnt2d

    denom = (G + r_lo) * (G + r_hi)
    W = jnp.where(cnt2d > 0, G * cnt2d / jnp.maximum(denom, 1.0), 0.0)
    W = W + jnp.where((G == 0.0) & (r_lo == 0.0) & (cnt2d > 0), 1.0, 0.0)
    ebar = sum2d / jnp.maximum(cnt2d, 1.0)
    loss = P * (1.0 / N) + jnp.sum(ebar * W)
    out_ref[0, 0] = loss


def _tc_finish(cnt, sm, sacc):
    return pl.pallas_call(
        _tc_finish_body,
        out_shape=jax.ShapeDtypeStruct((1, 1), jnp.float32),
        in_specs=[pl.BlockSpec(memory_space=pltpu.VMEM)] * 3,
        out_specs=pl.BlockSpec(memory_space=pltpu.SMEM),
    )(cnt, sm, sacc)


def kernel(outputs, masks):
    x = outputs.reshape(-1)
    m = masks.reshape(-1).astype(jnp.int32)
    cnt, sm, sacc = _sc_histogram()(x, m)
    return _tc_finish(cnt, sm, sacc).reshape(())


# speedup vs baseline: 85.6095x; 3.1389x over previous
import functools

import jax
import jax.numpy as jnp
from jax import lax
from jax.experimental import pallas as pl
from jax.experimental.pallas import tpu as pltpu
from jax.experimental.pallas import tpu_sc as plsc

L = 16
NC, NS = 2, 16
NTEC = NC * NS
K = 2048
NEL = 16 * 512 * 512
PER = NEL // NTEC
CHUNK = 4096
NCH = PER // CHUNK
NV = CHUNK // L
UNROLL = 8


def _sc_histogram_body(x_hbm, m_hbm, cnt_out, sum_out, s_out,
                       xb0, mb0, xb1, mb1, cnt_h, sum_h,
                       red_c, red_s, stmp, sx0, sm0, sx1, sm1):
    wid = lax.axis_index("s") * NC + lax.axis_index("c")
    lane = lax.iota(jnp.int32, L)
    lanebase = lane * K
    zeros16 = jnp.zeros((L,), jnp.float32)
    ones16 = jnp.ones((L,), jnp.float32)

    @plsc.parallel_loop(0, K, step=1, unroll=8)
    def _zero(i):
        cnt_h[pl.ds(i * L, L)] = zeros16
        sum_h[pl.ds(i * L, L)] = zeros16

    base0 = wid * PER

    def start_fetch(c, xb, mb, sx, sm_):
        pltpu.async_copy(x_hbm.at[pl.ds(base0 + c * CHUNK, CHUNK)], xb, sx)
        pltpu.async_copy(m_hbm.at[pl.ds(base0 + c * CHUNK, CHUNK)], mb, sm_)

    def wait_fetch(c, xb, mb, sx, sm_):
        pltpu.make_async_copy(
            x_hbm.at[pl.ds(base0 + c * CHUNK, CHUNK)], xb, sx).wait()
        pltpu.make_async_copy(
            m_hbm.at[pl.ds(base0 + c * CHUNK, CHUNK)], mb, sm_).wait()

    def process(xb, mb, acc):
        @plsc.parallel_loop(0, NV, step=1, unroll=UNROLL, carry=acc)
        def vbody(j, a):
            x = xb[pl.ds(j * L, L)]
            mi = mb[pl.ds(j * L, L)]
            s = 1.0 / (1.0 + jnp.exp(-x))
            b = jnp.minimum((s * float(K)).astype(jnp.int32), K - 1)
            neg = mi == 0
            plsc.addupdate_scatter(cnt_h, [lanebase + b], ones16, mask=neg)
            plsc.addupdate_scatter(sum_h, [lanebase + b], 1.0 + s, mask=neg)
            return a + s
        return vbody

    start_fetch(0, xb0, mb0, sx0, sm0)
    start_fetch(1, xb1, mb1, sx1, sm1)

    def chunk_body(i, acc):
        c0 = 2 * i
        wait_fetch(c0, xb0, mb0, sx0, sm0)
        acc = process(xb0, mb0, acc)

        @pl.when(c0 + 2 < NCH)
        def _():
            start_fetch(c0 + 2, xb0, mb0, sx0, sm0)

        wait_fetch(c0 + 1, xb1, mb1, sx1, sm1)
        acc = process(xb1, mb1, acc)

        @pl.when(c0 + 3 < NCH)
        def _():
            start_fetch(c0 + 3, xb1, mb1, sx1, sm1)

        return acc

    accs = lax.fori_loop(0, NCH // 2, chunk_body, zeros16)

    @plsc.parallel_loop(0, K // L, step=1, unroll=4)
    def _reduce(i):
        ac = cnt_h[pl.ds(i * L, L)]
        as_ = sum_h[pl.ds(i * L, L)]
        for l in range(1, L):
            ac = ac + cnt_h[pl.ds(l * K + i * L, L)]
            as_ = as_ + sum_h[pl.ds(l * K + i * L, L)]
        red_c[pl.ds(i * L, L)] = ac
        red_s[pl.ds(i * L, L)] = as_

    stmp[...] = accs
    pltpu.sync_copy(red_c, cnt_out.at[wid])
    pltpu.sync_copy(red_s, sum_out.at[wid])
    pltpu.sync_copy(stmp, s_out.at[wid])


@functools.cache
def _sc_histogram():
    return pl.kernel(
        _sc_histogram_body,
        out_type=(
            jax.ShapeDtypeStruct((NTEC, K), jnp.float32),
            jax.ShapeDtypeStruct((NTEC, K), jnp.float32),
            jax.ShapeDtypeStruct((NTEC, L), jnp.float32),
        ),
        mesh=plsc.VectorSubcoreMesh(
            core_axis_name="c", subcore_axis_name="s",
            num_cores=NC, num_subcores=NS),
        compiler_params=pltpu.CompilerParams(needs_layout_passes=False),
        scratch_types=[
            pltpu.VMEM((CHUNK,), jnp.float32),
            pltpu.VMEM((CHUNK,), jnp.int32),
            pltpu.VMEM((CHUNK,), jnp.float32),
            pltpu.VMEM((CHUNK,), jnp.int32),
            pltpu.VMEM((L * K,), jnp.float32),
            pltpu.VMEM((L * K,), jnp.float32),
            pltpu.VMEM((K,), jnp.float32),
            pltpu.VMEM((K,), jnp.float32),
            pltpu.VMEM((L,), jnp.float32),
            pltpu.SemaphoreType.DMA,
            pltpu.SemaphoreType.DMA,
            pltpu.SemaphoreType.DMA,
            pltpu.SemaphoreType.DMA,
        ],
    )


def _tc_finish_body(cnt_ref, sum_ref, s_ref, out_ref):
    cnt1 = jnp.sum(cnt_ref[...], axis=0, keepdims=True)
    sum1 = jnp.sum(sum_ref[...], axis=0, keepdims=True)
    R, C = K // 128, 128
    cnt2d = jnp.concatenate([cnt1[:, i * C:(i + 1) * C] for i in range(R)],
                            axis=0)
    sum2d = jnp.concatenate([sum1[:, i * C:(i + 1) * C] for i in range(R)],
                            axis=0)
    N = float(NEL)
    M = jnp.sum(cnt2d)
    Stot = jnp.sum(sum2d)
    Sall = jnp.sum(s_ref[...])
    G = N - M
    P = N - Sall - 2.0 * M + Stot

    kk = lax.broadcasted_iota(jnp.int32, (C, C), 0)
    jj = lax.broadcasted_iota(jnp.int32, (C, C), 1)
    A = (kk >= jj).astype(jnp.float32)
    inrow = jnp.dot(cnt2d, A, preferred_element_type=jnp.float32)
    rowsum = jnp.sum(cnt2d, axis=1, keepdims=True)
    rr = lax.broadcasted_iota(jnp.int32, (R, R), 0)
    cc = lax.broadcasted_iota(jnp.int32, (R, R), 1)
    B = (cc > rr).astype(jnp.float32)
    ra = jnp.dot(B, rowsum, preferred_element_type=jnp.float32)
    r_hi = inrow + ra
    r_lo = r_hi - cnt2d

    denom = (G + r_lo) * (G + r_hi)
    W = jnp.where(cnt2d > 0, G * cnt2d / jnp.maximum(denom, 1.0), 0.0)
    W = W + jnp.where((G == 0.0) & (r_lo == 0.0) & (cnt2d > 0), 1.0, 0.0)
    ebar = sum2d / jnp.maximum(cnt2d, 1.0)
    loss = P * (1.0 / N) + jnp.sum(ebar * W)
    out_ref[0, 0] = loss


def _tc_finish(cnt, sm, sacc):
    return pl.pallas_call(
        _tc_finish_body,
        out_shape=jax.ShapeDtypeStruct((1, 1), jnp.float32),
        in_specs=[pl.BlockSpec(memory_space=pltpu.VMEM)] * 3,
        out_specs=pl.BlockSpec(memory_space=pltpu.SMEM),
    )(cnt, sm, sacc)


def kernel(outputs, masks):
    x = outputs.reshape(-1)
    m = masks.reshape(-1).astype(jnp.int32)
    cnt, sm, sacc = _sc_histogram()(x, m)
    return _tc_finish(cnt, sm, sacc).reshape(())


# speedup vs baseline: 112.4289x; 1.3133x over previous
import functools

import jax
import jax.numpy as jnp
from jax import lax
from jax.experimental import pallas as pl
from jax.experimental.pallas import tpu as pltpu
from jax.experimental.pallas import tpu_sc as plsc

L = 16
NC, NS = 2, 16
NTEC = NC * NS
K = 2048
NEL = 16 * 512 * 512
NROW, NCOL = 8192, 512
ROWS_PER_TEC = NROW // NTEC
CROWS = 8
CHUNK = CROWS * NCOL
NCH = ROWS_PER_TEC // CROWS
NJ = NCOL // L
UNROLL = 2


def _sc_histogram_body(x_hbm, m_hbm, cnt_out, sum_out, s_out,
                       xb0, mb0, xb1, mb1, cnt_h, sum_h,
                       red_c, red_s, stmp, sx0, sm0, sx1, sm1):
    wid = lax.axis_index("s") * NC + lax.axis_index("c")
    lane = lax.iota(jnp.int32, L)
    lanebase = lane * K
    zeros16 = jnp.zeros((L,), jnp.float32)
    ones16 = jnp.ones((L,), jnp.float32)

    @plsc.parallel_loop(0, K, step=1, unroll=8)
    def _zero(i):
        cnt_h[pl.ds(i * L, L)] = zeros16
        sum_h[pl.ds(i * L, L)] = zeros16

    base0 = wid * ROWS_PER_TEC

    def start_fetch(c, xb, mb, sx, sm_):
        pltpu.async_copy(x_hbm.at[pl.ds(base0 + c * CROWS, CROWS)], xb, sx)
        pltpu.async_copy(m_hbm.at[pl.ds(base0 + c * CROWS, CROWS)], mb, sm_)

    def wait_fetch(c, xb, mb, sx, sm_):
        pltpu.make_async_copy(
            x_hbm.at[pl.ds(base0 + c * CROWS, CROWS)], xb, sx).wait()
        pltpu.make_async_copy(
            m_hbm.at[pl.ds(base0 + c * CROWS, CROWS)], mb, sm_).wait()

    def process(xb, mb, acc):
        @plsc.parallel_loop(0, NJ, step=1, unroll=UNROLL, carry=acc)
        def vbody(j, a):
            for r in range(CROWS):
                x = xb[r, pl.ds(j * L, L)]
                mi = mb[r, pl.ds(j * L, L)]
                s = 1.0 / (1.0 + jnp.exp(-x))
                b = jnp.minimum((s * float(K)).astype(jnp.int32), K - 1)
                neg = mi == 0
                plsc.addupdate_scatter(cnt_h, [lanebase + b], ones16,
                                       mask=neg)
                plsc.addupdate_scatter(sum_h, [lanebase + b], 1.0 + s,
                                       mask=neg)
                a = a + s
            return a
        return vbody

    start_fetch(0, xb0, mb0, sx0, sm0)
    start_fetch(1, xb1, mb1, sx1, sm1)

    def chunk_body(i, acc):
        c0 = 2 * i
        wait_fetch(c0, xb0, mb0, sx0, sm0)
        acc = process(xb0, mb0, acc)

        @pl.when(c0 + 2 < NCH)
        def _():
            start_fetch(c0 + 2, xb0, mb0, sx0, sm0)

        wait_fetch(c0 + 1, xb1, mb1, sx1, sm1)
        acc = process(xb1, mb1, acc)

        @pl.when(c0 + 3 < NCH)
        def _():
            start_fetch(c0 + 3, xb1, mb1, sx1, sm1)

        return acc

    accs = lax.fori_loop(0, NCH // 2, chunk_body, zeros16)

    @plsc.parallel_loop(0, K // L, step=1, unroll=4)
    def _reduce(i):
        ac = cnt_h[pl.ds(i * L, L)]
        as_ = sum_h[pl.ds(i * L, L)]
        for l in range(1, L):
            ac = ac + cnt_h[pl.ds(l * K + i * L, L)]
            as_ = as_ + sum_h[pl.ds(l * K + i * L, L)]
        red_c[pl.ds(i * L, L)] = ac
        red_s[pl.ds(i * L, L)] = as_

    stmp[...] = accs
    pltpu.sync_copy(red_c, cnt_out.at[wid])
    pltpu.sync_copy(red_s, sum_out.at[wid])
    pltpu.sync_copy(stmp, s_out.at[wid])


@functools.cache
def _sc_histogram():
    return pl.kernel(
        _sc_histogram_body,
        out_type=(
            jax.ShapeDtypeStruct((NTEC, K), jnp.float32),
            jax.ShapeDtypeStruct((NTEC, K), jnp.float32),
            jax.ShapeDtypeStruct((NTEC, L), jnp.float32),
        ),
        mesh=plsc.VectorSubcoreMesh(
            core_axis_name="c", subcore_axis_name="s",
            num_cores=NC, num_subcores=NS),
        compiler_params=pltpu.CompilerParams(needs_layout_passes=False),
        scratch_types=[
            pltpu.VMEM((CROWS, NCOL), jnp.float32),
            pltpu.VMEM((CROWS, NCOL), jnp.int32),
            pltpu.VMEM((CROWS, NCOL), jnp.float32),
            pltpu.VMEM((CROWS, NCOL), jnp.int32),
            pltpu.VMEM((L * K,), jnp.float32),
            pltpu.VMEM((L * K,), jnp.float32),
            pltpu.VMEM((K,), jnp.float32),
            pltpu.VMEM((K,), jnp.float32),
            pltpu.VMEM((L,), jnp.float32),
            pltpu.SemaphoreType.DMA,
            pltpu.SemaphoreType.DMA,
            pltpu.SemaphoreType.DMA,
            pltpu.SemaphoreType.DMA,
        ],
    )


def _tc_finish_body(cnt_ref, sum_ref, s_ref, out_ref):
    cnt1 = jnp.sum(cnt_ref[...], axis=0, keepdims=True)
    sum1 = jnp.sum(sum_ref[...], axis=0, keepdims=True)
    R, C = K // 128, 128
    cnt2d = jnp.concatenate([cnt1[:, i * C:(i + 1) * C] for i in range(R)],
                            axis=0)
    sum2d = jnp.concatenate([sum1[:, i * C:(i + 1) * C] for i in range(R)],
                            axis=0)
    N = float(NEL)
    M = jnp.sum(cnt2d)
    Stot = jnp.sum(sum2d)
    Sall = jnp.sum(s_ref[...])
    G = N - M
    P = N - Sall - 2.0 * M + Stot

    kk = lax.broadcasted_iota(jnp.int32, (C, C), 0)
    jj = lax.broadcasted_iota(jnp.int32, (C, C), 1)
    A = (kk >= jj).astype(jnp.float32)
    inrow = jnp.dot(cnt2d, A, preferred_element_type=jnp.float32)
    rowsum = jnp.sum(cnt2d, axis=1, keepdims=True)
    rr = lax.broadcasted_iota(jnp.int32, (R, R), 0)
    cc = lax.broadcasted_iota(jnp.int32, (R, R), 1)
    B = (cc > rr).astype(jnp.float32)
    ra = jnp.dot(B, rowsum, preferred_element_type=jnp.float32)
    r_hi = inrow + ra
    r_lo = r_hi - cnt2d

    denom = (G + r_lo) * (G + r_hi)
    W = jnp.where(cnt2d > 0, G * cnt2d / jnp.maximum(denom, 1.0), 0.0)
    W = W + jnp.where((G == 0.0) & (r_lo == 0.0) & (cnt2d > 0), 1.0, 0.0)
    ebar = sum2d / jnp.maximum(cnt2d, 1.0)
    loss = P * (1.0 / N) + jnp.sum(ebar * W)
    out_ref[0, 0] = loss


def _tc_finish(cnt, sm, sacc):
    return pl.pallas_call(
        _tc_finish_body,
        out_shape=jax.ShapeDtypeStruct((1, 1), jnp.float32),
        in_specs=[pl.BlockSpec(memory_space=pltpu.VMEM)] * 3,
        out_specs=pl.BlockSpec(memory_space=pltpu.SMEM),
    )(cnt, sm, sacc)


def kernel(outputs, masks):
    x = outputs.reshape(NROW, NCOL)
    m = masks.reshape(NROW, NCOL).astype(jnp.int32)
    cnt, sm, sacc = _sc_histogram()(x, m)
    return _tc_finish(cnt, sm, sacc).reshape(())


# speedup vs baseline: 120.2367x; 1.0694x over previous
import functools

import jax
import jax.numpy as jnp
from jax import lax
from jax.experimental import pallas as pl
from jax.experimental.pallas import tpu as pltpu
from jax.experimental.pallas import tpu_sc as plsc

L = 16
NC, NS = 2, 16
NTEC = NC * NS
K = 2048
NEL = 16 * 512 * 512
NROW, NCOL = 8192, 512
ROWS_PER_TEC = NROW // NTEC
CROWS = 8
CHUNK = CROWS * NCOL
NCH = ROWS_PER_TEC // CROWS
NJ = NCOL // L
UNROLL = 2


def _sc_histogram_body(x_hbm, m_hbm, cnt_out, sum_out, s_out,
                       xb0, mb0, xb1, mb1, cnt_h, sum_h,
                       red_c, red_s, stmp, sx0, sm0, sx1, sm1):
    wid = lax.axis_index("s") * NC + lax.axis_index("c")
    lane = lax.iota(jnp.int32, L)
    lanebase = lane * K - 0x3F800
    zeros16 = jnp.zeros((L,), jnp.float32)
    ones16 = jnp.ones((L,), jnp.float32)

    @plsc.parallel_loop(0, K, step=1, unroll=8)
    def _zero(i):
        cnt_h[pl.ds(i * L, L)] = zeros16
        sum_h[pl.ds(i * L, L)] = zeros16

    base0 = wid * ROWS_PER_TEC

    def start_fetch(c, xb, mb, sx, sm_):
        pltpu.async_copy(x_hbm.at[pl.ds(base0 + c * CROWS, CROWS)], xb, sx)
        pltpu.async_copy(m_hbm.at[pl.ds(base0 + c * CROWS, CROWS)], mb, sm_)

    def wait_fetch(c, xb, mb, sx, sm_):
        pltpu.make_async_copy(
            x_hbm.at[pl.ds(base0 + c * CROWS, CROWS)], xb, sx).wait()
        pltpu.make_async_copy(
            m_hbm.at[pl.ds(base0 + c * CROWS, CROWS)], mb, sm_).wait()

    def process(xb, mb, acc):
        @plsc.parallel_loop(0, NJ, step=1, unroll=UNROLL, carry=acc)
        def vbody(j, a):
            for r in range(CROWS):
                x = xb[r, pl.ds(j * L, L)]
                mi = mb[r, pl.ds(j * L, L)]
                s = 1.0 / (1.0 + jnp.exp(-x))
                e = jnp.minimum(1.0 + s, 1.9999999)
                idx = (plsc.bitcast(e, jnp.int32) >> 12) + lanebase
                neg = mi == 0
                plsc.addupdate_scatter(cnt_h, [idx], ones16, mask=neg)
                plsc.addupdate_scatter(sum_h, [idx], e, mask=neg)
                a = a + s
            return a
        return vbody

    start_fetch(0, xb0, mb0, sx0, sm0)
    start_fetch(1, xb1, mb1, sx1, sm1)

    def chunk_body(i, acc):
        c0 = 2 * i
        wait_fetch(c0, xb0, mb0, sx0, sm0)
        acc = process(xb0, mb0, acc)

        @pl.when(c0 + 2 < NCH)
        def _():
            start_fetch(c0 + 2, xb0, mb0, sx0, sm0)

        wait_fetch(c0 + 1, xb1, mb1, sx1, sm1)
        acc = process(xb1, mb1, acc)

        @pl.when(c0 + 3 < NCH)
        def _():
            start_fetch(c0 + 3, xb1, mb1, sx1, sm1)

        return acc

    accs = lax.fori_loop(0, NCH // 2, chunk_body, zeros16)

    @plsc.parallel_loop(0, K // L, step=1, unroll=4)
    def _reduce(i):
        ac = cnt_h[pl.ds(i * L, L)]
        as_ = sum_h[pl.ds(i * L, L)]
        for l in range(1, L):
            ac = ac + cnt_h[pl.ds(l * K + i * L, L)]
            as_ = as_ + sum_h[pl.ds(l * K + i * L, L)]
        red_c[pl.ds(i * L, L)] = ac
        red_s[pl.ds(i * L, L)] = as_

    stmp[...] = accs
    pltpu.sync_copy(red_c, cnt_out.at[wid])
    pltpu.sync_copy(red_s, sum_out.at[wid])
    pltpu.sync_copy(stmp, s_out.at[wid])


@functools.cache
def _sc_histogram():
    return pl.kernel(
        _sc_histogram_body,
        out_type=(
            jax.ShapeDtypeStruct((NTEC, K), jnp.float32),
            jax.ShapeDtypeStruct((NTEC, K), jnp.float32),
            jax.ShapeDtypeStruct((NTEC, L), jnp.float32),
        ),
        mesh=plsc.VectorSubcoreMesh(
            core_axis_name="c", subcore_axis_name="s",
            num_cores=NC, num_subcores=NS),
        compiler_params=pltpu.CompilerParams(needs_layout_passes=False),
        scratch_types=[
            pltpu.VMEM((CROWS, NCOL), jnp.float32),
            pltpu.VMEM((CROWS, NCOL), jnp.int32),
            pltpu.VMEM((CROWS, NCOL), jnp.float32),
            pltpu.VMEM((CROWS, NCOL), jnp.int32),
            pltpu.VMEM((L * K,), jnp.float32),
            pltpu.VMEM((L * K,), jnp.float32),
            pltpu.VMEM((K,), jnp.float32),
            pltpu.VMEM((K,), jnp.float32),
            pltpu.VMEM((L,), jnp.float32),
            pltpu.SemaphoreType.DMA,
            pltpu.SemaphoreType.DMA,
            pltpu.SemaphoreType.DMA,
            pltpu.SemaphoreType.DMA,
        ],
    )


def _tc_finish_body(cnt_ref, sum_ref, s_ref, out_ref):
    cnt1 = jnp.sum(cnt_ref[...], axis=0, keepdims=True)
    sum1 = jnp.sum(sum_ref[...], axis=0, keepdims=True)
    R, C = K // 128, 128
    cnt2d = jnp.concatenate([cnt1[:, i * C:(i + 1) * C] for i in range(R)],
                            axis=0)
    sum2d = jnp.concatenate([sum1[:, i * C:(i + 1) * C] for i in range(R)],
                            axis=0)
    N = float(NEL)
    M = jnp.sum(cnt2d)
    Stot = jnp.sum(sum2d)
    Sall = jnp.sum(s_ref[...])
    G = N - M
    P = N - Sall - 2.0 * M + Stot

    kk = lax.broadcasted_iota(jnp.int32, (C, C), 0)
    jj = lax.broadcasted_iota(jnp.int32, (C, C), 1)
    A = (kk >= jj).astype(jnp.float32)
    inrow = jnp.dot(cnt2d, A, preferred_element_type=jnp.float32)
    rowsum = jnp.sum(cnt2d, axis=1, keepdims=True)
    rr = lax.broadcasted_iota(jnp.int32, (R, R), 0)
    cc = lax.broadcasted_iota(jnp.int32, (R, R), 1)
    B = (cc > rr).astype(jnp.float32)
    ra = jnp.dot(B, rowsum, preferred_element_type=jnp.float32)
    r_hi = inrow + ra
    r_lo = r_hi - cnt2d

    denom = (G + r_lo) * (G + r_hi)
    W = jnp.where(cnt2d > 0, G * cnt2d / jnp.maximum(denom, 1.0), 0.0)
    W = W + jnp.where((G == 0.0) & (r_lo == 0.0) & (cnt2d > 0), 1.0, 0.0)
    ebar = sum2d / jnp.maximum(cnt2d, 1.0)
    loss = P * (1.0 / N) + jnp.sum(ebar * W)
    out_ref[0, 0] = loss


def _tc_finish(cnt, sm, sacc):
    return pl.pallas_call(
        _tc_finish_body,
        out_shape=jax.ShapeDtypeStruct((1, 1), jnp.float32),
        in_specs=[pl.BlockSpec(memory_space=pltpu.VMEM)] * 3,
        out_specs=pl.BlockSpec(memory_space=pltpu.SMEM),
    )(cnt, sm, sacc)


def kernel(outputs, masks):
    x = outputs.reshape(NROW, NCOL)
    m = masks.reshape(NROW, NCOL).astype(jnp.int32)
    cnt, sm, sacc = _sc_histogram()(x, m)
    return _tc_finish(cnt, sm, sacc).reshape(())


# speedup vs baseline: 131.5748x; 1.0943x over previous
import functools

import jax
import jax.numpy as jnp
from jax import lax
from jax.experimental import pallas as pl
from jax.experimental.pallas import tpu as pltpu
from jax.experimental.pallas import tpu_sc as plsc

L = 16
NC, NS = 2, 16
NTEC = NC * NS
K = 2048
NEL = 16 * 512 * 512
NROW, NCOL = 8192, 512
ROWS_PER_TEC = NROW // NTEC
CROWS = 8
CHUNK = CROWS * NCOL
NCH = ROWS_PER_TEC // CROWS
NJ = NCOL // L
UNROLL = 2


def _sc_histogram_body(x_hbm, m_hbm, cnt_out, sum_out, s_out,
                       xb0, mb0, xb1, mb1, cnt_h, sum_h,
                       stmp, sx0, sm0, sx1, sm1):
    wid = lax.axis_index("s") * NC + lax.axis_index("c")
    zeros16 = jnp.zeros((L,), jnp.float32)
    ones16 = jnp.ones((L,), jnp.float32)

    @plsc.parallel_loop(0, K // L, step=1, unroll=8)
    def _zero(i):
        cnt_h[pl.ds(i * L, L)] = zeros16
        sum_h[pl.ds(i * L, L)] = zeros16

    base0 = wid * ROWS_PER_TEC

    def start_fetch(c, xb, mb, sx, sm_):
        pltpu.async_copy(x_hbm.at[pl.ds(base0 + c * CROWS, CROWS)], xb, sx)
        pltpu.async_copy(m_hbm.at[pl.ds(base0 + c * CROWS, CROWS)], mb, sm_)

    def wait_fetch(c, xb, mb, sx, sm_):
        pltpu.make_async_copy(
            x_hbm.at[pl.ds(base0 + c * CROWS, CROWS)], xb, sx).wait()
        pltpu.make_async_copy(
            m_hbm.at[pl.ds(base0 + c * CROWS, CROWS)], mb, sm_).wait()

    def process(xb, mb, acc):
        @plsc.parallel_loop(0, NJ, step=1, unroll=UNROLL, carry=acc)
        def vbody(j, a):
            for r in range(CROWS):
                x = xb[r, pl.ds(j * L, L)]
                mi = mb[r, pl.ds(j * L, L)]
                s = 1.0 / (1.0 + jnp.exp(-x))
                e = jnp.minimum(1.0 + s, 1.9999999)
                idx = (plsc.bitcast(e, jnp.int32) >> 12) - 0x3F800
                neg = mi == 0
                plsc.addupdate_scatter(cnt_h, [idx], ones16, mask=neg)
                plsc.addupdate_scatter(sum_h, [idx], e, mask=neg)
                a = a + s
            return a
        return vbody

    start_fetch(0, xb0, mb0, sx0, sm0)
    start_fetch(1, xb1, mb1, sx1, sm1)

    def chunk_body(i, acc):
        c0 = 2 * i
        wait_fetch(c0, xb0, mb0, sx0, sm0)
        acc = process(xb0, mb0, acc)

        @pl.when(c0 + 2 < NCH)
        def _():
            start_fetch(c0 + 2, xb0, mb0, sx0, sm0)

        wait_fetch(c0 + 1, xb1, mb1, sx1, sm1)
        acc = process(xb1, mb1, acc)

        @pl.when(c0 + 3 < NCH)
        def _():
            start_fetch(c0 + 3, xb1, mb1, sx1, sm1)

        return acc

    accs = lax.fori_loop(0, NCH // 2, chunk_body, zeros16)

    stmp[...] = accs
    pltpu.sync_copy(cnt_h, cnt_out.at[wid])
    pltpu.sync_copy(sum_h, sum_out.at[wid])
    pltpu.sync_copy(stmp, s_out.at[wid])


@functools.cache
def _sc_histogram():
    return pl.kernel(
        _sc_histogram_body,
        out_type=(
            jax.ShapeDtypeStruct((NTEC, K), jnp.float32),
            jax.ShapeDtypeStruct((NTEC, K), jnp.float32),
            jax.ShapeDtypeStruct((NTEC, L), jnp.float32),
        ),
        mesh=plsc.VectorSubcoreMesh(
            core_axis_name="c", subcore_axis_name="s",
            num_cores=NC, num_subcores=NS),
        compiler_params=pltpu.CompilerParams(needs_layout_passes=False),
        scratch_types=[
            pltpu.VMEM((CROWS, NCOL), jnp.float32),
            pltpu.VMEM((CROWS, NCOL), jnp.int32),
            pltpu.VMEM((CROWS, NCOL), jnp.float32),
            pltpu.VMEM((CROWS, NCOL), jnp.int32),
            pltpu.VMEM((K,), jnp.float32),
            pltpu.VMEM((K,), jnp.float32),
            pltpu.VMEM((L,), jnp.float32),
            pltpu.SemaphoreType.DMA,
            pltpu.SemaphoreType.DMA,
            pltpu.SemaphoreType.DMA,
            pltpu.SemaphoreType.DMA,
        ],
    )


def _tc_finish_body(cnt_ref, sum_ref, s_ref, out_ref):
    cnt1 = jnp.sum(cnt_ref[...], axis=0, keepdims=True)
    sum1 = jnp.sum(sum_ref[...], axis=0, keepdims=True)
    R, C = K // 128, 128
    cnt2d = jnp.concatenate([cnt1[:, i * C:(i + 1) * C] for i in range(R)],
                            axis=0)
    sum2d = jnp.concatenate([sum1[:, i * C:(i + 1) * C] for i in range(R)],
                            axis=0)
    N = float(NEL)
    M = jnp.sum(cnt2d)
    Stot = jnp.sum(sum2d)
    Sall = jnp.sum(s_ref[...])
    G = N - M
    P = N - Sall - 2.0 * M + Stot

    kk = lax.broadcasted_iota(jnp.int32, (C, C), 0)
    jj = lax.broadcasted_iota(jnp.int32, (C, C), 1)
    A = (kk >= jj).astype(jnp.float32)
    inrow = jnp.dot(cnt2d, A, preferred_element_type=jnp.float32)
    rowsum = jnp.sum(cnt2d, axis=1, keepdims=True)
    rr = lax.broadcasted_iota(jnp.int32, (R, R), 0)
    cc = lax.broadcasted_iota(jnp.int32, (R, R), 1)
    B = (cc > rr).astype(jnp.float32)
    ra = jnp.dot(B, rowsum, preferred_element_type=jnp.float32)
    r_hi = inrow + ra
    r_lo = r_hi - cnt2d

    denom = (G + r_lo) * (G + r_hi)
    W = jnp.where(cnt2d > 0, G * cnt2d / jnp.maximum(denom, 1.0), 0.0)
    W = W + jnp.where((G == 0.0) & (r_lo == 0.0) & (cnt2d > 0), 1.0, 0.0)
    ebar = sum2d / jnp.maximum(cnt2d, 1.0)
    loss = P * (1.0 / N) + jnp.sum(ebar * W)
    out_ref[0, 0] = loss


def _tc_finish(cnt, sm, sacc):
    return pl.pallas_call(
        _tc_finish_body,
        out_shape=jax.ShapeDtypeStruct((1, 1), jnp.float32),
        in_specs=[pl.BlockSpec(memory_space=pltpu.VMEM)] * 3,
        out_specs=pl.BlockSpec(memory_space=pltpu.SMEM),
    )(cnt, sm, sacc)


def kernel(outputs, masks):
    x = outputs.reshape(NROW, NCOL)
    m = masks.reshape(NROW, NCOL).astype(jnp.int32)
    cnt, sm, sacc = _sc_histogram()(x, m)
    return _tc_finish(cnt, sm, sacc).reshape(())
